# Initial kernel scaffold; baseline (speedup 1.0000x reference)
import jax
import jax.numpy as jnp
from jax.experimental import pallas as pl


def kernel(patch_embeddings):
    raise NotImplementedError("write your pallas kernel here")


# speedup vs baseline: 5.0289x; 5.0289x over previous
import functools

import jax
import jax.numpy as jnp
import numpy as np
from jax import lax
from jax.experimental import pallas as pl
from jax.experimental.pallas import tpu as pltpu
from jax.experimental.pallas import tpu_sc as plsc

_B, _N, _D = 64, 1024, 768
_NUM_MASKED = int(0.75 * _N)
_NUM_UNMASKED = _N - _NUM_MASKED
_NW = 32
_ROWS_W = _B // _NW
_GROWS_W = _B * _NUM_UNMASKED // _NW
_CHUNK = 64
_NCHUNK = _GROWS_W // _CHUNK


def _rotl32(x, d):
    return ((x << np.uint32(d)) | (x >> np.uint32(32 - d))).astype(np.uint32)


def _threefry2x32(k0, k1, x0, x1):
    ks = [np.uint32(k0), np.uint32(k1),
          np.uint32(np.uint32(k0) ^ np.uint32(k1) ^ np.uint32(0x1BD11BDA))]
    rot = [(13, 15, 26, 6), (17, 29, 16, 24)]
    x = [(x0 + ks[0]).astype(np.uint32), (x1 + ks[1]).astype(np.uint32)]
    for i in range(5):
        for r in rot[i % 2]:
            x[0] = (x[0] + x[1]).astype(np.uint32)
            x[1] = _rotl32(x[1], r)
            x[1] = (x[1] ^ x[0]).astype(np.uint32)
        x[0] = (x[0] + ks[(i + 1) % 3]).astype(np.uint32)
        x[1] = (x[1] + ks[(i + 2) % 3] + np.uint32(i + 1)).astype(np.uint32)
    return x


@functools.cache
def _host_indices():
    seed = 1
    size = _B * _N
    k0 = np.uint32((seed >> 32) & 0xFFFFFFFF)
    k1 = np.uint32(seed & 0xFFFFFFFF)
    i = np.arange(size, dtype=np.uint64)
    c_hi = (i >> np.uint64(32)).astype(np.uint32)
    c_lo = (i & np.uint64(0xFFFFFFFF)).astype(np.uint32)
    y = _threefry2x32(k0, k1, c_hi, c_lo)
    bits = (y[0] ^ y[1]).reshape(_B, _N)
    noise = ((bits >> np.uint32(9)) | np.uint32(0x3F800000)).view(np.float32) - np.float32(1.0)
    noise = np.maximum(np.float32(0.0), noise)
    perm = np.argsort(noise, axis=1, kind="stable").astype(np.int32)
    masked = perm[:, :_NUM_MASKED].copy()
    unmasked = perm[:, _NUM_MASKED:].copy()
    mask = np.zeros((_B, _N), dtype=np.int32)
    np.put_along_axis(mask, masked, 1, axis=1)
    return masked, unmasked, mask


def _sc_body(x_ref, uidx_ref, midx_ref, mask_in_ref,
             out_ref, mask_ref, midx_out_ref, uidx_out_ref,
             uidx_v, fidx_v, midx_v, mask_v, buf0, buf1, sem0, sem1):
    w = lax.axis_index("s") * 2 + lax.axis_index("c")
    b0 = w * _ROWS_W

    pltpu.sync_copy(uidx_ref.at[pl.ds(b0 * _NUM_UNMASKED, _GROWS_W)], uidx_v)
    for r in range(_ROWS_W):
        base = (b0 + r) * _N
        for c in range(_NUM_UNMASKED // 16):
            o = r * _NUM_UNMASKED + c * 16
            fidx_v[pl.ds(o, 16)] = uidx_v[pl.ds(o, 16)] + base

    bufs = (buf0, buf1)
    sems = (sem0, sem1)
    cps = [None, None]
    cps[0] = pltpu.async_copy(x_ref.at[fidx_v.at[pl.ds(0, _CHUNK)]], buf0, sem0)

    pltpu.sync_copy(mask_in_ref.at[pl.ds(b0 * _N, _ROWS_W * _N)], mask_v)
    pltpu.sync_copy(mask_v, mask_ref.at[pl.ds(b0 * _N, _ROWS_W * _N)])
    pltpu.sync_copy(midx_ref.at[pl.ds(b0 * _NUM_MASKED, _ROWS_W * _NUM_MASKED)], midx_v)
    pltpu.sync_copy(midx_v, midx_out_ref.at[pl.ds(b0 * _NUM_MASKED, _ROWS_W * _NUM_MASKED)])
    pltpu.sync_copy(uidx_v, uidx_out_ref.at[pl.ds(b0 * _NUM_UNMASKED, _GROWS_W)])

    for g in range(_NCHUNK):
        cur, nxt = g & 1, (g + 1) & 1
        if g + 1 < _NCHUNK:
            cps[nxt] = pltpu.async_copy(
                x_ref.at[fidx_v.at[pl.ds((g + 1) * _CHUNK, _CHUNK)]],
                bufs[nxt], sems[nxt])
        cps[cur].wait()
        pltpu.sync_copy(bufs[cur], out_ref.at[pl.ds(w * _GROWS_W + g * _CHUNK, _CHUNK)])


@functools.cache
def _sc_call():
    mesh = plsc.VectorSubcoreMesh(core_axis_name="c", subcore_axis_name="s")
    return pl.kernel(
        _sc_body,
        mesh=mesh,
        out_type=[
            jax.ShapeDtypeStruct((_B * _NUM_UNMASKED, _D), jnp.float32),
            jax.ShapeDtypeStruct((_B * _N,), jnp.int32),
            jax.ShapeDtypeStruct((_B * _NUM_MASKED,), jnp.int32),
            jax.ShapeDtypeStruct((_B * _NUM_UNMASKED,), jnp.int32),
        ],
        scratch_types=[
            pltpu.VMEM((_GROWS_W,), jnp.int32),
            pltpu.VMEM((_GROWS_W,), jnp.int32),
            pltpu.VMEM((_ROWS_W * _NUM_MASKED,), jnp.int32),
            pltpu.VMEM((_ROWS_W * _N,), jnp.int32),
            pltpu.VMEM((_CHUNK, _D), jnp.float32),
            pltpu.VMEM((_CHUNK, _D), jnp.float32),
            pltpu.SemaphoreType.DMA,
            pltpu.SemaphoreType.DMA,
        ],
    )


def kernel(patch_embeddings):
    masked_np, unmasked_np, mask_np = _host_indices()
    midx_flat = jnp.asarray(masked_np.reshape(-1), dtype=jnp.int32)
    uidx_flat = jnp.asarray(unmasked_np.reshape(-1), dtype=jnp.int32)
    mask_flat = jnp.asarray(mask_np.reshape(-1), dtype=jnp.int32)
    x2d = patch_embeddings.reshape(_B * _N, _D)
    out2d, mask_i32, midx_o, uidx_o = _sc_call()(x2d, uidx_flat, midx_flat, mask_flat)
    unmasked_patches_only = out2d.reshape(_B, _NUM_UNMASKED, _D)
    bool_mask = mask_i32.reshape(_B, _N).astype(bool)
    masked_indices = midx_o.reshape(_B, _NUM_MASKED)
    unmasked_indices = uidx_o.reshape(_B, _NUM_UNMASKED)
    return (unmasked_patches_only, bool_mask, masked_indices, unmasked_indices)


# speedup vs baseline: 5.0531x; 1.0048x over previous
import functools

import jax
import jax.numpy as jnp
import numpy as np
from jax import lax
from jax.experimental import pallas as pl
from jax.experimental.pallas import tpu as pltpu
from jax.experimental.pallas import tpu_sc as plsc

_B, _N, _D = 64, 1024, 768
_NUM_MASKED = int(0.75 * _N)
_NUM_UNMASKED = _N - _NUM_MASKED
_NW = 32
_ROWS_W = _B // _NW
_GROWS_W = _B * _NUM_UNMASKED // _NW
_CHUNK = 32
_NCHUNK = _GROWS_W // _CHUNK
_NBUF = 4


def _rotl32(x, d):
    return ((x << np.uint32(d)) | (x >> np.uint32(32 - d))).astype(np.uint32)


def _threefry2x32(k0, k1, x0, x1):
    ks = [np.uint32(k0), np.uint32(k1),
          np.uint32(np.uint32(k0) ^ np.uint32(k1) ^ np.uint32(0x1BD11BDA))]
    rot = [(13, 15, 26, 6), (17, 29, 16, 24)]
    x = [(x0 + ks[0]).astype(np.uint32), (x1 + ks[1]).astype(np.uint32)]
    for i in range(5):
        for r in rot[i % 2]:
            x[0] = (x[0] + x[1]).astype(np.uint32)
            x[1] = _rotl32(x[1], r)
            x[1] = (x[1] ^ x[0]).astype(np.uint32)
        x[0] = (x[0] + ks[(i + 1) % 3]).astype(np.uint32)
        x[1] = (x[1] + ks[(i + 2) % 3] + np.uint32(i + 1)).astype(np.uint32)
    return x


@functools.cache
def _host_indices():
    seed = 1
    size = _B * _N
    k0 = np.uint32((seed >> 32) & 0xFFFFFFFF)
    k1 = np.uint32(seed & 0xFFFFFFFF)
    i = np.arange(size, dtype=np.uint64)
    c_hi = (i >> np.uint64(32)).astype(np.uint32)
    c_lo = (i & np.uint64(0xFFFFFFFF)).astype(np.uint32)
    y = _threefry2x32(k0, k1, c_hi, c_lo)
    bits = (y[0] ^ y[1]).reshape(_B, _N)
    noise = ((bits >> np.uint32(9)) | np.uint32(0x3F800000)).view(np.float32) - np.float32(1.0)
    noise = np.maximum(np.float32(0.0), noise)
    perm = np.argsort(noise, axis=1, kind="stable").astype(np.int32)
    masked = perm[:, :_NUM_MASKED].copy()
    unmasked = perm[:, _NUM_MASKED:].copy()
    mask = np.zeros((_B, _N), dtype=np.int32)
    np.put_along_axis(mask, masked, 1, axis=1)
    return masked, unmasked, mask


def _sc_body(x_ref, uidx_ref, midx_ref, mask_in_ref,
             out_ref, mask_ref, midx_out_ref, uidx_out_ref,
             uidx_v, fidx_v, midx_v, mask_v,
             buf0, buf1, buf2, buf3, gs0, gs1, gs2, gs3, ws0, ws1, ws2, ws3):
    w = lax.axis_index("s") * 2 + lax.axis_index("c")
    b0 = w * _ROWS_W
    bufs = (buf0, buf1, buf2, buf3)
    gsems = (gs0, gs1, gs2, gs3)
    wsems = (ws0, ws1, ws2, ws3)

    pltpu.sync_copy(uidx_ref.at[pl.ds(b0 * _NUM_UNMASKED, _GROWS_W)], uidx_v)
    for r in range(_ROWS_W):
        base = (b0 + r) * _N
        for c in range(_NUM_UNMASKED // 16):
            o = r * _NUM_UNMASKED + c * 16
            fidx_v[pl.ds(o, 16)] = uidx_v[pl.ds(o, 16)] + base

    def _gather(g, b):
        return pltpu.async_copy(
            x_ref.at[fidx_v.at[pl.ds(g * _CHUNK, _CHUNK)]], bufs[b], gsems[b])

    def _put(g, b):
        return pltpu.async_copy(
            bufs[b], out_ref.at[pl.ds(w * _GROWS_W + g * _CHUNK, _CHUNK)], wsems[b])

    cps_g = [_gather(b, b) for b in range(_NBUF)]
    cps_w = [None] * _NBUF

    pltpu.sync_copy(mask_in_ref.at[pl.ds(b0 * _N, _ROWS_W * _N)], mask_v)
    pltpu.sync_copy(mask_v, mask_ref.at[pl.ds(b0 * _N, _ROWS_W * _N)])
    pltpu.sync_copy(midx_ref.at[pl.ds(b0 * _NUM_MASKED, _ROWS_W * _NUM_MASKED)], midx_v)
    pltpu.sync_copy(midx_v, midx_out_ref.at[pl.ds(b0 * _NUM_MASKED, _ROWS_W * _NUM_MASKED)])
    pltpu.sync_copy(uidx_v, uidx_out_ref.at[pl.ds(b0 * _NUM_UNMASKED, _GROWS_W)])

    for g in range(_NCHUNK):
        b = g % _NBUF
        cps_g[b].wait()
        cps_w[b] = _put(g, b)
        if g + _NBUF < _NCHUNK:
            cps_w[b].wait()
            cps_g[b] = _gather(g + _NBUF, b)
    for g in range(max(0, _NCHUNK - _NBUF), _NCHUNK):
        cps_w[g % _NBUF].wait()


@functools.cache
def _sc_call():
    mesh = plsc.VectorSubcoreMesh(core_axis_name="c", subcore_axis_name="s")
    return pl.kernel(
        _sc_body,
        mesh=mesh,
        out_type=[
            jax.ShapeDtypeStruct((_B * _NUM_UNMASKED, _D), jnp.float32),
            jax.ShapeDtypeStruct((_B * _N,), jnp.int32),
            jax.ShapeDtypeStruct((_B * _NUM_MASKED,), jnp.int32),
            jax.ShapeDtypeStruct((_B * _NUM_UNMASKED,), jnp.int32),
        ],
        scratch_types=(
            [
                pltpu.VMEM((_GROWS_W,), jnp.int32),
                pltpu.VMEM((_GROWS_W,), jnp.int32),
                pltpu.VMEM((_ROWS_W * _NUM_MASKED,), jnp.int32),
                pltpu.VMEM((_ROWS_W * _N,), jnp.int32),
            ]
            + [pltpu.VMEM((_CHUNK, _D), jnp.float32) for _ in range(_NBUF)]
            + [pltpu.SemaphoreType.DMA for _ in range(2 * _NBUF)]
        ),
    )


def kernel(patch_embeddings):
    masked_np, unmasked_np, mask_np = _host_indices()
    midx_flat = jnp.asarray(masked_np.reshape(-1), dtype=jnp.int32)
    uidx_flat = jnp.asarray(unmasked_np.reshape(-1), dtype=jnp.int32)
    mask_flat = jnp.asarray(mask_np.reshape(-1), dtype=jnp.int32)
    x2d = patch_embeddings.reshape(_B * _N, _D)
    out2d, mask_i32, midx_o, uidx_o = _sc_call()(x2d, uidx_flat, midx_flat, mask_flat)
    unmasked_patches_only = out2d.reshape(_B, _NUM_UNMASKED, _D)
    bool_mask = mask_i32.reshape(_B, _N).astype(bool)
    masked_indices = midx_o.reshape(_B, _NUM_MASKED)
    unmasked_indices = uidx_o.reshape(_B, _NUM_UNMASKED)
    return (unmasked_patches_only, bool_mask, masked_indices, unmasked_indices)


# speedup vs baseline: 5.0621x; 1.0018x over previous
import functools

import jax
import jax.numpy as jnp
import numpy as np
from jax import lax
from jax.experimental import pallas as pl
from jax.experimental.pallas import tpu as pltpu
from jax.experimental.pallas import tpu_sc as plsc

_B, _N, _D = 64, 1024, 768
_NUM_MASKED = int(0.75 * _N)
_NUM_UNMASKED = _N - _NUM_MASKED
_NW = 32
_ROWS_W = _B // _NW
_GROWS_W = _B * _NUM_UNMASKED // _NW
_CHUNK = 32
_NCHUNK = _GROWS_W // _CHUNK
_NBUF = 4


def _rotl32(x, d):
    return ((x << np.uint32(d)) | (x >> np.uint32(32 - d))).astype(np.uint32)


def _threefry2x32(k0, k1, x0, x1):
    ks = [np.uint32(k0), np.uint32(k1),
          np.uint32(np.uint32(k0) ^ np.uint32(k1) ^ np.uint32(0x1BD11BDA))]
    rot = [(13, 15, 26, 6), (17, 29, 16, 24)]
    x = [(x0 + ks[0]).astype(np.uint32), (x1 + ks[1]).astype(np.uint32)]
    for i in range(5):
        for r in rot[i % 2]:
            x[0] = (x[0] + x[1]).astype(np.uint32)
            x[1] = _rotl32(x[1], r)
            x[1] = (x[1] ^ x[0]).astype(np.uint32)
        x[0] = (x[0] + ks[(i + 1) % 3]).astype(np.uint32)
        x[1] = (x[1] + ks[(i + 2) % 3] + np.uint32(i + 1)).astype(np.uint32)
    return x


@functools.cache
def _host_indices():
    seed = 1
    size = _B * _N
    k0 = np.uint32((seed >> 32) & 0xFFFFFFFF)
    k1 = np.uint32(seed & 0xFFFFFFFF)
    i = np.arange(size, dtype=np.uint64)
    c_hi = (i >> np.uint64(32)).astype(np.uint32)
    c_lo = (i & np.uint64(0xFFFFFFFF)).astype(np.uint32)
    y = _threefry2x32(k0, k1, c_hi, c_lo)
    bits = (y[0] ^ y[1]).reshape(_B, _N)
    noise = ((bits >> np.uint32(9)) | np.uint32(0x3F800000)).view(np.float32) - np.float32(1.0)
    noise = np.maximum(np.float32(0.0), noise)
    perm = np.argsort(noise, axis=1, kind="stable").astype(np.int32)
    masked = perm[:, :_NUM_MASKED].copy()
    unmasked = perm[:, _NUM_MASKED:].copy()
    mask = np.zeros((_B, _N), dtype=np.bool_)
    np.put_along_axis(mask, masked, True, axis=1)
    return masked, unmasked, mask


def _sc_body(x_ref, uidx_ref, midx_ref, mask_in_ref,
             out_ref, mask_ref, midx_out_ref, uidx_out_ref,
             uidx_v, fidx_v, midx_v, mask_v,
             buf0, buf1, buf2, buf3, gs0, gs1, gs2, gs3, ws0, ws1, ws2, ws3):
    w = lax.axis_index("s") * 2 + lax.axis_index("c")
    b0 = w * _ROWS_W
    bufs = (buf0, buf1, buf2, buf3)
    gsems = (gs0, gs1, gs2, gs3)
    wsems = (ws0, ws1, ws2, ws3)

    pltpu.sync_copy(uidx_ref.at[pl.ds(b0 * _NUM_UNMASKED, _GROWS_W)], uidx_v)
    for r in range(_ROWS_W):
        base = (b0 + r) * _N
        for c in range(_NUM_UNMASKED // 16):
            o = r * _NUM_UNMASKED + c * 16
            fidx_v[pl.ds(o, 16)] = uidx_v[pl.ds(o, 16)] + base

    def _gather(g, b):
        return pltpu.async_copy(
            x_ref.at[fidx_v.at[pl.ds(g * _CHUNK, _CHUNK)]], bufs[b], gsems[b])

    def _put(g, b):
        return pltpu.async_copy(
            bufs[b], out_ref.at[pl.ds(w * _GROWS_W + g * _CHUNK, _CHUNK)], wsems[b])

    cps_g = [_gather(b, b) for b in range(_NBUF)]
    cps_w = [None] * _NBUF

    pltpu.sync_copy(mask_in_ref.at[pl.ds(b0 * _N, _ROWS_W * _N)], mask_v)
    pltpu.sync_copy(mask_v, mask_ref.at[pl.ds(b0 * _N, _ROWS_W * _N)])
    pltpu.sync_copy(midx_ref.at[pl.ds(b0 * _NUM_MASKED, _ROWS_W * _NUM_MASKED)], midx_v)
    pltpu.sync_copy(midx_v, midx_out_ref.at[pl.ds(b0 * _NUM_MASKED, _ROWS_W * _NUM_MASKED)])
    pltpu.sync_copy(uidx_v, uidx_out_ref.at[pl.ds(b0 * _NUM_UNMASKED, _GROWS_W)])

    for g in range(_NCHUNK):
        b = g % _NBUF
        cps_g[b].wait()
        cps_w[b] = _put(g, b)
        if g + _NBUF < _NCHUNK:
            cps_w[b].wait()
            cps_g[b] = _gather(g + _NBUF, b)
    for g in range(max(0, _NCHUNK - _NBUF), _NCHUNK):
        cps_w[g % _NBUF].wait()


@functools.cache
def _sc_call():
    mesh = plsc.VectorSubcoreMesh(core_axis_name="c", subcore_axis_name="s")
    return pl.kernel(
        _sc_body,
        mesh=mesh,
        out_type=[
            jax.ShapeDtypeStruct((_B * _NUM_UNMASKED, _D), jnp.float32),
            jax.ShapeDtypeStruct((_B * _N,), jnp.bool_),
            jax.ShapeDtypeStruct((_B * _NUM_MASKED,), jnp.int32),
            jax.ShapeDtypeStruct((_B * _NUM_UNMASKED,), jnp.int32),
        ],
        scratch_types=(
            [
                pltpu.VMEM((_GROWS_W,), jnp.int32),
                pltpu.VMEM((_GROWS_W,), jnp.int32),
                pltpu.VMEM((_ROWS_W * _NUM_MASKED,), jnp.int32),
                pltpu.VMEM((_ROWS_W * _N,), jnp.bool_),
            ]
            + [pltpu.VMEM((_CHUNK, _D), jnp.float32) for _ in range(_NBUF)]
            + [pltpu.SemaphoreType.DMA for _ in range(2 * _NBUF)]
        ),
    )


def kernel(patch_embeddings):
    masked_np, unmasked_np, mask_np = _host_indices()
    midx_flat = jnp.asarray(masked_np.reshape(-1), dtype=jnp.int32)
    uidx_flat = jnp.asarray(unmasked_np.reshape(-1), dtype=jnp.int32)
    mask_flat = jnp.asarray(mask_np.reshape(-1))
    x2d = patch_embeddings.reshape(_B * _N, _D)
    out2d, mask_b, midx_o, uidx_o = _sc_call()(x2d, uidx_flat, midx_flat, mask_flat)
    unmasked_patches_only = out2d.reshape(_B, _NUM_UNMASKED, _D)
    bool_mask = mask_b.reshape(_B, _N)
    masked_indices = midx_o.reshape(_B, _NUM_MASKED)
    unmasked_indices = uidx_o.reshape(_B, _NUM_UNMASKED)
    return (unmasked_patches_only, bool_mask, masked_indices, unmasked_indices)


# speedup vs baseline: 5.5286x; 1.0922x over previous
import functools

import jax
import jax.numpy as jnp
import numpy as np
from jax import lax
from jax.experimental import pallas as pl
from jax.experimental.pallas import tpu as pltpu
from jax.experimental.pallas import tpu_sc as plsc

_B, _N, _D = 64, 1024, 768
_NUM_MASKED = int(0.75 * _N)
_NUM_UNMASKED = _N - _NUM_MASKED
_NW = 32
_GROWS_W = _B * _NUM_UNMASKED // _NW
_CHUNK = 32
_NCHUNK = _GROWS_W // _CHUNK
_NBUF = 4


def _rotl32(x, d):
    return ((x << np.uint32(d)) | (x >> np.uint32(32 - d))).astype(np.uint32)


def _threefry2x32(k0, k1, x0, x1):
    ks = [np.uint32(k0), np.uint32(k1),
          np.uint32(np.uint32(k0) ^ np.uint32(k1) ^ np.uint32(0x1BD11BDA))]
    rot = [(13, 15, 26, 6), (17, 29, 16, 24)]
    x = [(x0 + ks[0]).astype(np.uint32), (x1 + ks[1]).astype(np.uint32)]
    for i in range(5):
        for r in rot[i % 2]:
            x[0] = (x[0] + x[1]).astype(np.uint32)
            x[1] = _rotl32(x[1], r)
            x[1] = (x[1] ^ x[0]).astype(np.uint32)
        x[0] = (x[0] + ks[(i + 1) % 3]).astype(np.uint32)
        x[1] = (x[1] + ks[(i + 2) % 3] + np.uint32(i + 1)).astype(np.uint32)
    return x


@functools.cache
def _host_indices():
    seed = 1
    size = _B * _N
    k0 = np.uint32((seed >> 32) & 0xFFFFFFFF)
    k1 = np.uint32(seed & 0xFFFFFFFF)
    i = np.arange(size, dtype=np.uint64)
    c_hi = (i >> np.uint64(32)).astype(np.uint32)
    c_lo = (i & np.uint64(0xFFFFFFFF)).astype(np.uint32)
    y = _threefry2x32(k0, k1, c_hi, c_lo)
    bits = (y[0] ^ y[1]).reshape(_B, _N)
    noise = ((bits >> np.uint32(9)) | np.uint32(0x3F800000)).view(np.float32) - np.float32(1.0)
    noise = np.maximum(np.float32(0.0), noise)
    perm = np.argsort(noise, axis=1, kind="stable").astype(np.int32)
    masked = perm[:, :_NUM_MASKED].copy()
    unmasked = perm[:, _NUM_MASKED:].copy()
    mask = np.zeros((_B, _N), dtype=np.bool_)
    np.put_along_axis(mask, masked, True, axis=1)
    flat_rows = (unmasked + np.arange(_B, dtype=np.int32)[:, None] * _N).reshape(-1)
    return masked, unmasked, mask, flat_rows


def _sc_body(x_ref, fidx_ref, out_ref, fidx_v,
             buf0, buf1, buf2, buf3, gs0, gs1, gs2, gs3, ws0, ws1, ws2, ws3):
    w = lax.axis_index("s") * 2 + lax.axis_index("c")
    bufs = (buf0, buf1, buf2, buf3)
    gsems = (gs0, gs1, gs2, gs3)
    wsems = (ws0, ws1, ws2, ws3)

    pltpu.sync_copy(fidx_ref.at[pl.ds(w * _GROWS_W, _GROWS_W)], fidx_v)

    def _gather(g, b):
        return pltpu.async_copy(
            x_ref.at[fidx_v.at[pl.ds(g * _CHUNK, _CHUNK)]], bufs[b], gsems[b])

    def _put(g, b):
        return pltpu.async_copy(
            bufs[b], out_ref.at[pl.ds(w * _GROWS_W + g * _CHUNK, _CHUNK)], wsems[b])

    cps_g = [_gather(b, b) for b in range(_NBUF)]
    cps_w = [None] * _NBUF

    for g in range(_NCHUNK):
        b = g % _NBUF
        cps_g[b].wait()
        cps_w[b] = _put(g, b)
        if g + _NBUF < _NCHUNK:
            cps_w[b].wait()
            cps_g[b] = _gather(g + _NBUF, b)
    for g in range(max(0, _NCHUNK - _NBUF), _NCHUNK):
        cps_w[g % _NBUF].wait()


@functools.cache
def _sc_call():
    mesh = plsc.VectorSubcoreMesh(core_axis_name="c", subcore_axis_name="s")
    return pl.kernel(
        _sc_body,
        mesh=mesh,
        out_type=jax.ShapeDtypeStruct((_B * _NUM_UNMASKED, _D), jnp.float32),
        scratch_types=(
            [pltpu.VMEM((_GROWS_W,), jnp.int32)]
            + [pltpu.VMEM((_CHUNK, _D), jnp.float32) for _ in range(_NBUF)]
            + [pltpu.SemaphoreType.DMA for _ in range(2 * _NBUF)]
        ),
    )


def kernel(patch_embeddings):
    masked_np, unmasked_np, mask_np, flat_rows_np = _host_indices()
    fidx_flat = jnp.asarray(flat_rows_np)
    x2d = patch_embeddings.reshape(_B * _N, _D)
    out2d = _sc_call()(x2d, fidx_flat)
    unmasked_patches_only = out2d.reshape(_B, _NUM_UNMASKED, _D)
    bool_mask = jnp.asarray(mask_np)
    masked_indices = jnp.asarray(masked_np)
    unmasked_indices = jnp.asarray(unmasked_np)
    return (unmasked_patches_only, bool_mask, masked_indices, unmasked_indices)
